# baseline (speedup 1.0000x reference)
import numpy as np

import jax
import jax.numpy as jnp
from jax import lax
from jax.experimental import pallas as pl
from jax.experimental.pallas import tpu as pltpu
from jax.experimental.pallas import tpu_sc as plsc

_N = 10000
_E = 320000
_D = 128
_G = 256
_NF = 9
_BN_EPS = 1e-5

_NPAD = 10240
_NCHUNK = _NPAD // 128
_NC, _NS = 2, 16
_NW = _NC * _NS
_EPW = _NPAD
_ECH = _EPW // 128
_EPAD = _NW * _EPW
_BLK = 512
_GRID = _NPAD // _BLK
_GP = 264
_STRIPE = _NPAD // _NS

_f32 = jnp.float32


def _sc_mesh():
    return plsc.VectorSubcoreMesh(core_axis_name="c", subcore_axis_name="s")


_DW = 128


def _deg_body(row3, onesc, zcol, out, rowv, onesv, deg_sh, dsem):
    cid = lax.axis_index("c")
    sid = lax.axis_index("s")
    wid = cid * _NS + sid
    for k in range(_STRIPE // 128):
        pltpu.sync_copy(zcol, deg_sh.at[pl.ds(sid * _STRIPE + k * 128, 128)])
    pltpu.sync_copy(onesc, onesv)
    pltpu.sync_copy(row3.at[wid], rowv)
    plsc.subcore_barrier()

    def body(j, _):
        pltpu.async_copy(onesv, deg_sh.at[rowv.at[j]], dsem, add=True)
        return ()

    lax.fori_loop(0, _ECH, body, ())

    def drain(j, _):
        pltpu.make_async_copy(onesv, deg_sh.at[pl.ds(0, 128)], dsem).wait()
        return ()

    lax.fori_loop(0, _ECH, drain, ())
    plsc.subcore_barrier()

    @pl.when(sid == 0)
    def _():
        pltpu.sync_copy(deg_sh, out.at[cid])


_s_deg = pl.kernel(
    _deg_body,
    out_type=jax.ShapeDtypeStruct((_NC, _NPAD, _DW), _f32),
    mesh=_sc_mesh(),
    scratch_types=[
        pltpu.VMEM((_ECH, 128), jnp.int32),
        pltpu.VMEM((128, _DW), _f32),
        pltpu.VMEM_SHARED((_NPAD, _DW), _f32),
        pltpu.SemaphoreType.DMA,
    ],
)


_W = 40
_RW = _ECH // _W


def _edge_body(g, row3, col3, z128, out,
               rw, cw, b0, b1, aggr_sh,
               g0, g1, s0, s1):
    cid = lax.axis_index("c")
    sid = lax.axis_index("s")
    wid = cid * _NS + sid
    bufs = [b0, b1]
    gsem = [g0, g1]
    ssem = [s0, s1]
    for k in range(_STRIPE // 128):
        pltpu.sync_copy(z128, aggr_sh.at[pl.ds(sid * _STRIPE + k * 128, 128)])
    plsc.subcore_barrier()

    for r in range(_RW):
        pltpu.sync_copy(row3.at[wid, pl.ds(r * _W, _W)], rw)
        pltpu.sync_copy(col3.at[wid, pl.ds(r * _W, _W)], cw)
        pltpu.async_copy(g.at[rw.at[0]], b0, g0)
        pltpu.async_copy(g.at[rw.at[1]], b1, g1)

        def pair(t, _):
            for b in (0, 1):
                k = 2 * t + b
                pltpu.make_async_copy(g.at[pl.ds(0, 128)], bufs[b],
                                      gsem[b]).wait()
                pltpu.async_copy(bufs[b], aggr_sh.at[cw.at[k]], ssem[b],
                                 add=True)
                pltpu.make_async_copy(bufs[b], aggr_sh.at[pl.ds(0, 128)],
                                      ssem[b]).wait()

                @pl.when(k + 2 < _W)
                def _():
                    pltpu.async_copy(g.at[rw.at[k + 2]], bufs[b], gsem[b])

            return ()

        lax.fori_loop(0, _W // 2, pair, ())

    plsc.subcore_barrier()
    for k in range(_STRIPE // 128):
        sl = pl.ds(sid * _STRIPE + k * 128, 128)
        pltpu.sync_copy(aggr_sh.at[sl], out.at[cid, sl, :])


_s_edge = pl.kernel(
    _edge_body,
    out_type=jax.ShapeDtypeStruct((_NC, _NPAD, _D), _f32),
    mesh=_sc_mesh(),
    scratch_types=[
        pltpu.VMEM((_W, 128), jnp.int32),
        pltpu.VMEM((_W, 128), jnp.int32),
        pltpu.VMEM((128, _D), _f32),
        pltpu.VMEM((128, _D), _f32),
        pltpu.VMEM_SHARED((_NPAD, _D), _f32),
        pltpu.SemaphoreType.DMA,
        pltpu.SemaphoreType.DMA,
        pltpu.SemaphoreType.DMA,
        pltpu.SemaphoreType.DMA,
    ],
)


def _pool_body(h3, batch3, z128, zcol, onesc, outp, outc,
               bv, rbuf, onesv, pool_sh, cnt_sh):
    cid = lax.axis_index("c")
    sid = lax.axis_index("s")
    wid = cid * _NS + sid

    @pl.when(sid == 0)
    def _():
        pltpu.sync_copy(z128, pool_sh.at[pl.ds(0, 128)])
        pltpu.sync_copy(z128, pool_sh.at[pl.ds(128, 128)])
        pltpu.sync_copy(z128.at[pl.ds(0, 8), :], pool_sh.at[pl.ds(256, 8)])

    @pl.when(sid == 1)
    def _():
        pltpu.sync_copy(zcol, cnt_sh.at[pl.ds(0, 128)])
        pltpu.sync_copy(zcol, cnt_sh.at[pl.ds(128, 128)])
        pltpu.sync_copy(zcol.at[pl.ds(0, 8), :], cnt_sh.at[pl.ds(256, 8)])

    pltpu.sync_copy(onesc, onesv)
    plsc.subcore_barrier()
    for k in range(3):
        c = wid + _NW * k

        @pl.when(c < _NCHUNK)
        def _():
            pltpu.sync_copy(batch3.at[c], bv.at[k])
            pltpu.sync_copy(h3.at[pl.ds(c * 128, 128), :], rbuf)
            pltpu.sync_copy(rbuf, pool_sh.at[bv.at[k]], add=True)
            pltpu.sync_copy(onesv, cnt_sh.at[bv.at[k]], add=True)

    plsc.subcore_barrier()

    @pl.when(sid == 0)
    def _():
        pltpu.sync_copy(pool_sh.at[pl.ds(0, _G)], outp.at[cid])

    @pl.when(sid == 1)
    def _():
        pltpu.sync_copy(cnt_sh.at[pl.ds(0, _G)], outc.at[cid])


_s_pool = pl.kernel(
    _pool_body,
    out_type=(
        jax.ShapeDtypeStruct((_NC, _G, _D), _f32),
        jax.ShapeDtypeStruct((_NC, _G, _DW), _f32),
    ),
    mesh=_sc_mesh(),
    scratch_types=[
        pltpu.VMEM((3, 128), jnp.int32),
        pltpu.VMEM((128, _D), _f32),
        pltpu.VMEM((128, _DW), _f32),
        pltpu.VMEM_SHARED((_GP, _D), _f32),
        pltpu.VMEM_SHARED((_GP, _DW), _f32),
    ],
)


def _t0a_body(xp, emb, W, b, hh_out):
    xb = xp[...]
    embv = emb[...]
    iot = lax.broadcasted_iota(jnp.int32, (_BLK, 128), 1)
    h0 = jnp.zeros((_BLK, _D), _f32)
    for i in range(_NF):
        oh = jnp.where(iot == xb[:, i:i + 1], 1.0, 0.0)
        h0 = h0 + lax.dot_general(oh, embv[i * 128:(i + 1) * 128],
                                  (((1,), (0,)), ((), ())),
                                  preferred_element_type=_f32)
    hh_out[...] = lax.dot_general(h0, W[...], (((1,), (1,)), ((), ())),
                                  preferred_element_type=_f32) + b[...]


_t0a = pl.pallas_call(
    _t0a_body,
    grid=(_GRID,),
    in_specs=[
        pl.BlockSpec((_BLK, 16), lambda i: (i, 0)),
        pl.BlockSpec((_NF * 128, _D), lambda i: (0, 0)),
        pl.BlockSpec((_D, _D), lambda i: (0, 0)),
        pl.BlockSpec((1, _D), lambda i: (0, 0)),
    ],
    out_specs=pl.BlockSpec((_BLK, _D), lambda i: (i, 0)),
    out_shape=jax.ShapeDtypeStruct((_NPAD, _D), _f32),
)


def _t0b_body(hhp, dp, root, g_out, self_out, deg_out, dis_out):
    degp = dp[...]
    deg = degp[0, :, 0:1] + degp[1, :, 0:1] + 1.0
    dis = lax.rsqrt(deg)
    hh = hhp[...]
    g_out[...] = dis * jnp.maximum(hh, 0.0)
    self_out[...] = jnp.maximum(hh + root[...], 0.0) / deg
    deg_out[...] = deg
    dis_out[...] = dis


_t0b = pl.pallas_call(
    _t0b_body,
    grid=(_GRID,),
    in_specs=[
        pl.BlockSpec((_BLK, _D), lambda i: (i, 0)),
        pl.BlockSpec((_NC, _BLK, _DW), lambda i: (0, i, 0)),
        pl.BlockSpec((1, _D), lambda i: (0, 0)),
    ],
    out_specs=[
        pl.BlockSpec((_BLK, _D), lambda i: (i, 0)),
        pl.BlockSpec((_BLK, _D), lambda i: (i, 0)),
        pl.BlockSpec((_BLK, 1), lambda i: (i, 0)),
        pl.BlockSpec((_BLK, 1), lambda i: (i, 0)),
    ],
    out_shape=[
        jax.ShapeDtypeStruct((_NPAD, _D), _f32),
        jax.ShapeDtypeStruct((_NPAD, _D), _f32),
        jax.ShapeDtypeStruct((_NPAD, 1), _f32),
        jax.ShapeDtypeStruct((_NPAD, 1), _f32),
    ],
)


def _tmid_body(ap, sp, dis, deg, gam, bet, W, b, root, g_out, self_out):
    a = ap[...]
    h = dis[...] * (a[0] + a[1]) + sp[...]
    h = gam[...] * h + bet[...]
    h = jnp.maximum(h, 0.0)
    hh = lax.dot_general(h, W[...], (((1,), (1,)), ((), ())),
                         preferred_element_type=_f32) + b[...]
    g_out[...] = dis[...] * jnp.maximum(hh, 0.0)
    self_out[...] = jnp.maximum(hh + root[...], 0.0) / deg[...]


_tmid = pl.pallas_call(
    _tmid_body,
    grid=(_GRID,),
    in_specs=[
        pl.BlockSpec((_NC, _BLK, _D), lambda i: (0, i, 0)),
        pl.BlockSpec((_BLK, _D), lambda i: (i, 0)),
        pl.BlockSpec((_BLK, 1), lambda i: (i, 0)),
        pl.BlockSpec((_BLK, 1), lambda i: (i, 0)),
        pl.BlockSpec((1, _D), lambda i: (0, 0)),
        pl.BlockSpec((1, _D), lambda i: (0, 0)),
        pl.BlockSpec((_D, _D), lambda i: (0, 0)),
        pl.BlockSpec((1, _D), lambda i: (0, 0)),
        pl.BlockSpec((1, _D), lambda i: (0, 0)),
    ],
    out_specs=[
        pl.BlockSpec((_BLK, _D), lambda i: (i, 0)),
        pl.BlockSpec((_BLK, _D), lambda i: (i, 0)),
    ],
    out_shape=[
        jax.ShapeDtypeStruct((_NPAD, _D), _f32),
        jax.ShapeDtypeStruct((_NPAD, _D), _f32),
    ],
)


def _t3_body(ap, sp, dis, h_out):
    a = ap[...]
    h_out[...] = dis[...] * (a[0] + a[1]) + sp[...]


_t3 = pl.pallas_call(
    _t3_body,
    grid=(_GRID,),
    in_specs=[
        pl.BlockSpec((_NC, _BLK, _D), lambda i: (0, i, 0)),
        pl.BlockSpec((_BLK, _D), lambda i: (i, 0)),
        pl.BlockSpec((_BLK, 1), lambda i: (i, 0)),
    ],
    out_specs=pl.BlockSpec((_BLK, _D), lambda i: (i, 0)),
    out_shape=jax.ShapeDtypeStruct((_NPAD, _D), _f32),
)


def _t4_body(pp, cc, W1, b1, W2, b2, o_out):
    p = pp[...]
    c = cc[...]
    cnt = jnp.maximum(c[0, :, 0:1] + c[1, :, 0:1], 1.0)
    pooled = jnp.maximum((p[0] + p[1]) / cnt, 0.0)
    o = lax.dot_general(pooled, W1[...], (((1,), (1,)), ((), ())),
                        preferred_element_type=_f32) + b1[...]
    o = jnp.maximum(o, 0.0)
    o_out[...] = lax.dot_general(o, W2[...], (((1,), (1,)), ((), ())),
                                 preferred_element_type=_f32) + b2[...]


_t4 = pl.pallas_call(
    _t4_body,
    out_shape=jax.ShapeDtypeStruct((_G, _D), _f32),
)


def kernel(x, edge_index, batch, atom_emb, lin_W, lin_b, root_emb,
           bn_gamma, bn_beta, W1, b1, W2, b2):
    xp = jnp.zeros((_NPAD, 16), jnp.int32).at[:_N, :_NF].set(x)
    emb = jnp.pad(atom_emb, ((0, 0), (0, 128 - atom_emb.shape[1]), (0, 0))
                  ).reshape(_NF * 128, _D)
    row = edge_index[0]
    col = edge_index[1]
    pad_e = _N + jnp.arange(_EPAD - _E, dtype=jnp.int32) % (_NPAD - _N)
    row3 = jnp.concatenate([row, pad_e]).reshape(_NW, _ECH, 128)
    col3 = jnp.concatenate([col, pad_e]).reshape(_NW, _ECH, 128)
    batch3 = jnp.concatenate(
        [batch, jnp.full((_NPAD - _N,), _G, jnp.int32)]).reshape(_NCHUNK, 128)
    onesc = jnp.ones((128, _DW), _f32)
    zcol = jnp.zeros((128, _DW), _f32)
    z128 = jnp.zeros((128, _D), _f32)
    bn_scale = 1.0 / np.sqrt(1.0 + _BN_EPS)

    deg_parts = _s_deg(row3, onesc, zcol)
    hh = _t0a(xp, emb, lin_W[0], lin_b[0][None])
    g, selfv, deg, dis = _t0b(hh, deg_parts, root_emb[0][None])
    for l in (0, 1):
        aggr = _s_edge(g, row3, col3, z128)
        g, selfv = _tmid(aggr, selfv, dis, deg,
                         (bn_gamma[l] * bn_scale)[None], bn_beta[l][None],
                         lin_W[l + 1], lin_b[l + 1][None],
                         root_emb[l + 1][None])
    aggr = _s_edge(g, row3, col3, z128)
    h3 = _t3(aggr, selfv, dis)
    pooled_p, cnt_p = _s_pool(h3, batch3, z128, zcol, onesc)
    return _t4(pooled_p, cnt_p, W1, b1[None], W2, b2[None])


# speedup vs baseline: 1.0018x; 1.0018x over previous
import numpy as np

import jax
import jax.numpy as jnp
from jax import lax
from jax.experimental import pallas as pl
from jax.experimental.pallas import tpu as pltpu
from jax.experimental.pallas import tpu_sc as plsc

_N = 10000
_E = 320000
_D = 128
_G = 256
_NF = 9
_BN_EPS = 1e-5

_NPAD = 10240
_NCHUNK = _NPAD // 128
_NC, _NS = 2, 16
_NW = _NC * _NS
_EPW = _NPAD
_ECH = _EPW // 128
_EPAD = _NW * _EPW
_BLK = 512
_GRID = _NPAD // _BLK
_GP = 264
_STRIPE = _NPAD // _NS

_f32 = jnp.float32


def _sc_mesh():
    return plsc.VectorSubcoreMesh(core_axis_name="c", subcore_axis_name="s")


_DW = 128


def _deg_body(row3, onesc, zcol, out, rowv, onesv, deg_sh, dsem):
    cid = lax.axis_index("c")
    sid = lax.axis_index("s")
    wid = cid * _NS + sid
    for k in range(_STRIPE // 128):
        pltpu.async_copy(zcol, deg_sh.at[pl.ds(sid * _STRIPE + k * 128, 128)],
                         dsem)
    pltpu.sync_copy(onesc, onesv)
    pltpu.sync_copy(row3.at[wid], rowv)
    for k in range(_STRIPE // 128):
        pltpu.make_async_copy(zcol, deg_sh.at[pl.ds(0, 128)], dsem).wait()
    plsc.subcore_barrier()

    def body(j, _):
        pltpu.async_copy(onesv, deg_sh.at[rowv.at[j]], dsem, add=True)
        return ()

    lax.fori_loop(0, _ECH, body, ())

    def drain(j, _):
        pltpu.make_async_copy(onesv, deg_sh.at[pl.ds(0, 128)], dsem).wait()
        return ()

    lax.fori_loop(0, _ECH, drain, ())
    plsc.subcore_barrier()

    @pl.when(sid == 0)
    def _():
        pltpu.sync_copy(deg_sh, out.at[cid])


_s_deg = pl.kernel(
    _deg_body,
    out_type=jax.ShapeDtypeStruct((_NC, _NPAD, _DW), _f32),
    mesh=_sc_mesh(),
    scratch_types=[
        pltpu.VMEM((_ECH, 128), jnp.int32),
        pltpu.VMEM((128, _DW), _f32),
        pltpu.VMEM_SHARED((_NPAD, _DW), _f32),
        pltpu.SemaphoreType.DMA,
    ],
)


_W = 40
_RW = _ECH // _W


def _edge_body(g, row3, col3, z128, out,
               rw, cw, b0, b1, aggr_sh,
               g0, g1, s0, s1):
    cid = lax.axis_index("c")
    sid = lax.axis_index("s")
    wid = cid * _NS + sid
    bufs = [b0, b1]
    gsem = [g0, g1]
    ssem = [s0, s1]
    for k in range(_STRIPE // 128):
        pltpu.async_copy(z128, aggr_sh.at[pl.ds(sid * _STRIPE + k * 128, 128)],
                         g0)
    for k in range(_STRIPE // 128):
        pltpu.make_async_copy(z128, aggr_sh.at[pl.ds(0, 128)], g0).wait()
    plsc.subcore_barrier()

    for r in range(_RW):
        pltpu.sync_copy(row3.at[wid, pl.ds(r * _W, _W)], rw)
        pltpu.sync_copy(col3.at[wid, pl.ds(r * _W, _W)], cw)
        pltpu.async_copy(g.at[rw.at[0]], b0, g0)
        pltpu.async_copy(g.at[rw.at[1]], b1, g1)

        def pair(t, _):
            for b in (0, 1):
                k = 2 * t + b
                pltpu.make_async_copy(g.at[pl.ds(0, 128)], bufs[b],
                                      gsem[b]).wait()
                pltpu.async_copy(bufs[b], aggr_sh.at[cw.at[k]], ssem[b],
                                 add=True)
                pltpu.make_async_copy(bufs[b], aggr_sh.at[pl.ds(0, 128)],
                                      ssem[b]).wait()

                @pl.when(k + 2 < _W)
                def _():
                    pltpu.async_copy(g.at[rw.at[k + 2]], bufs[b], gsem[b])

            return ()

        lax.fori_loop(0, _W // 2, pair, ())

    plsc.subcore_barrier()
    for k in range(_STRIPE // 128):
        sl = pl.ds(sid * _STRIPE + k * 128, 128)
        pltpu.async_copy(aggr_sh.at[sl], out.at[cid, sl, :], g0)
    for k in range(_STRIPE // 128):
        pltpu.make_async_copy(aggr_sh.at[pl.ds(0, 128)],
                              out.at[cid, pl.ds(0, 128), :], g0).wait()


_s_edge = pl.kernel(
    _edge_body,
    out_type=jax.ShapeDtypeStruct((_NC, _NPAD, _D), _f32),
    mesh=_sc_mesh(),
    scratch_types=[
        pltpu.VMEM((_W, 128), jnp.int32),
        pltpu.VMEM((_W, 128), jnp.int32),
        pltpu.VMEM((128, _D), _f32),
        pltpu.VMEM((128, _D), _f32),
        pltpu.VMEM_SHARED((_NPAD, _D), _f32),
        pltpu.SemaphoreType.DMA,
        pltpu.SemaphoreType.DMA,
        pltpu.SemaphoreType.DMA,
        pltpu.SemaphoreType.DMA,
    ],
)


def _pool_body(h3, batch3, z128, zcol, onesc, outp, outc,
               bv, rbuf, onesv, pool_sh, cnt_sh):
    cid = lax.axis_index("c")
    sid = lax.axis_index("s")
    wid = cid * _NS + sid

    @pl.when(sid == 0)
    def _():
        pltpu.sync_copy(z128, pool_sh.at[pl.ds(0, 128)])
        pltpu.sync_copy(z128, pool_sh.at[pl.ds(128, 128)])
        pltpu.sync_copy(z128.at[pl.ds(0, 8), :], pool_sh.at[pl.ds(256, 8)])

    @pl.when(sid == 1)
    def _():
        pltpu.sync_copy(zcol, cnt_sh.at[pl.ds(0, 128)])
        pltpu.sync_copy(zcol, cnt_sh.at[pl.ds(128, 128)])
        pltpu.sync_copy(zcol.at[pl.ds(0, 8), :], cnt_sh.at[pl.ds(256, 8)])

    pltpu.sync_copy(onesc, onesv)
    plsc.subcore_barrier()
    for k in range(3):
        c = wid + _NW * k

        @pl.when(c < _NCHUNK)
        def _():
            pltpu.sync_copy(batch3.at[c], bv.at[k])
            pltpu.sync_copy(h3.at[pl.ds(c * 128, 128), :], rbuf)
            pltpu.sync_copy(rbuf, pool_sh.at[bv.at[k]], add=True)
            pltpu.sync_copy(onesv, cnt_sh.at[bv.at[k]], add=True)

    plsc.subcore_barrier()

    @pl.when(sid == 0)
    def _():
        pltpu.sync_copy(pool_sh.at[pl.ds(0, _G)], outp.at[cid])

    @pl.when(sid == 1)
    def _():
        pltpu.sync_copy(cnt_sh.at[pl.ds(0, _G)], outc.at[cid])


_s_pool = pl.kernel(
    _pool_body,
    out_type=(
        jax.ShapeDtypeStruct((_NC, _G, _D), _f32),
        jax.ShapeDtypeStruct((_NC, _G, _DW), _f32),
    ),
    mesh=_sc_mesh(),
    scratch_types=[
        pltpu.VMEM((3, 128), jnp.int32),
        pltpu.VMEM((128, _D), _f32),
        pltpu.VMEM((128, _DW), _f32),
        pltpu.VMEM_SHARED((_GP, _D), _f32),
        pltpu.VMEM_SHARED((_GP, _DW), _f32),
    ],
)


def _t0a_body(xp, emb, W, b, hh_out):
    xb = xp[...]
    embv = emb[...]
    iot = lax.broadcasted_iota(jnp.int32, (_BLK, 128), 1)
    h0 = jnp.zeros((_BLK, _D), _f32)
    for i in range(_NF):
        oh = jnp.where(iot == xb[:, i:i + 1], 1.0, 0.0)
        h0 = h0 + lax.dot_general(oh, embv[i * 128:(i + 1) * 128],
                                  (((1,), (0,)), ((), ())),
                                  preferred_element_type=_f32)
    hh_out[...] = lax.dot_general(h0, W[...], (((1,), (1,)), ((), ())),
                                  preferred_element_type=_f32) + b[...]


_t0a = pl.pallas_call(
    _t0a_body,
    grid=(_GRID,),
    in_specs=[
        pl.BlockSpec((_BLK, 16), lambda i: (i, 0)),
        pl.BlockSpec((_NF * 128, _D), lambda i: (0, 0)),
        pl.BlockSpec((_D, _D), lambda i: (0, 0)),
        pl.BlockSpec((1, _D), lambda i: (0, 0)),
    ],
    out_specs=pl.BlockSpec((_BLK, _D), lambda i: (i, 0)),
    out_shape=jax.ShapeDtypeStruct((_NPAD, _D), _f32),
)


def _t0b_body(hhp, dp, root, g_out, self_out, deg_out, dis_out):
    degp = dp[...]
    deg = degp[0, :, 0:1] + degp[1, :, 0:1] + 1.0
    dis = lax.rsqrt(deg)
    hh = hhp[...]
    g_out[...] = dis * jnp.maximum(hh, 0.0)
    self_out[...] = jnp.maximum(hh + root[...], 0.0) / deg
    deg_out[...] = deg
    dis_out[...] = dis


_t0b = pl.pallas_call(
    _t0b_body,
    grid=(_GRID,),
    in_specs=[
        pl.BlockSpec((_BLK, _D), lambda i: (i, 0)),
        pl.BlockSpec((_NC, _BLK, _DW), lambda i: (0, i, 0)),
        pl.BlockSpec((1, _D), lambda i: (0, 0)),
    ],
    out_specs=[
        pl.BlockSpec((_BLK, _D), lambda i: (i, 0)),
        pl.BlockSpec((_BLK, _D), lambda i: (i, 0)),
        pl.BlockSpec((_BLK, 1), lambda i: (i, 0)),
        pl.BlockSpec((_BLK, 1), lambda i: (i, 0)),
    ],
    out_shape=[
        jax.ShapeDtypeStruct((_NPAD, _D), _f32),
        jax.ShapeDtypeStruct((_NPAD, _D), _f32),
        jax.ShapeDtypeStruct((_NPAD, 1), _f32),
        jax.ShapeDtypeStruct((_NPAD, 1), _f32),
    ],
)


def _tmid_body(ap, sp, dis, deg, gam, bet, W, b, root, g_out, self_out):
    a = ap[...]
    h = dis[...] * (a[0] + a[1]) + sp[...]
    h = gam[...] * h + bet[...]
    h = jnp.maximum(h, 0.0)
    hh = lax.dot_general(h, W[...], (((1,), (1,)), ((), ())),
                         preferred_element_type=_f32) + b[...]
    g_out[...] = dis[...] * jnp.maximum(hh, 0.0)
    self_out[...] = jnp.maximum(hh + root[...], 0.0) / deg[...]


_tmid = pl.pallas_call(
    _tmid_body,
    grid=(_GRID,),
    in_specs=[
        pl.BlockSpec((_NC, _BLK, _D), lambda i: (0, i, 0)),
        pl.BlockSpec((_BLK, _D), lambda i: (i, 0)),
        pl.BlockSpec((_BLK, 1), lambda i: (i, 0)),
        pl.BlockSpec((_BLK, 1), lambda i: (i, 0)),
        pl.BlockSpec((1, _D), lambda i: (0, 0)),
        pl.BlockSpec((1, _D), lambda i: (0, 0)),
        pl.BlockSpec((_D, _D), lambda i: (0, 0)),
        pl.BlockSpec((1, _D), lambda i: (0, 0)),
        pl.BlockSpec((1, _D), lambda i: (0, 0)),
    ],
    out_specs=[
        pl.BlockSpec((_BLK, _D), lambda i: (i, 0)),
        pl.BlockSpec((_BLK, _D), lambda i: (i, 0)),
    ],
    out_shape=[
        jax.ShapeDtypeStruct((_NPAD, _D), _f32),
        jax.ShapeDtypeStruct((_NPAD, _D), _f32),
    ],
)


def _t3_body(ap, sp, dis, h_out):
    a = ap[...]
    h_out[...] = dis[...] * (a[0] + a[1]) + sp[...]


_t3 = pl.pallas_call(
    _t3_body,
    grid=(_GRID,),
    in_specs=[
        pl.BlockSpec((_NC, _BLK, _D), lambda i: (0, i, 0)),
        pl.BlockSpec((_BLK, _D), lambda i: (i, 0)),
        pl.BlockSpec((_BLK, 1), lambda i: (i, 0)),
    ],
    out_specs=pl.BlockSpec((_BLK, _D), lambda i: (i, 0)),
    out_shape=jax.ShapeDtypeStruct((_NPAD, _D), _f32),
)


def _t4_body(pp, cc, W1, b1, W2, b2, o_out):
    p = pp[...]
    c = cc[...]
    cnt = jnp.maximum(c[0, :, 0:1] + c[1, :, 0:1], 1.0)
    pooled = jnp.maximum((p[0] + p[1]) / cnt, 0.0)
    o = lax.dot_general(pooled, W1[...], (((1,), (1,)), ((), ())),
                        preferred_element_type=_f32) + b1[...]
    o = jnp.maximum(o, 0.0)
    o_out[...] = lax.dot_general(o, W2[...], (((1,), (1,)), ((), ())),
                                 preferred_element_type=_f32) + b2[...]


_t4 = pl.pallas_call(
    _t4_body,
    out_shape=jax.ShapeDtypeStruct((_G, _D), _f32),
)


def kernel(x, edge_index, batch, atom_emb, lin_W, lin_b, root_emb,
           bn_gamma, bn_beta, W1, b1, W2, b2):
    xp = jnp.zeros((_NPAD, 16), jnp.int32).at[:_N, :_NF].set(x)
    emb = jnp.pad(atom_emb, ((0, 0), (0, 128 - atom_emb.shape[1]), (0, 0))
                  ).reshape(_NF * 128, _D)
    row = edge_index[0]
    col = edge_index[1]
    pad_e = _N + jnp.arange(_EPAD - _E, dtype=jnp.int32) % (_NPAD - _N)
    row3 = jnp.concatenate([row, pad_e]).reshape(_NW, _ECH, 128)
    col3 = jnp.concatenate([col, pad_e]).reshape(_NW, _ECH, 128)
    batch3 = jnp.concatenate(
        [batch, jnp.full((_NPAD - _N,), _G, jnp.int32)]).reshape(_NCHUNK, 128)
    onesc = jnp.ones((128, _DW), _f32)
    zcol = jnp.zeros((128, _DW), _f32)
    z128 = jnp.zeros((128, _D), _f32)
    bn_scale = 1.0 / np.sqrt(1.0 + _BN_EPS)

    deg_parts = _s_deg(row3, onesc, zcol)
    hh = _t0a(xp, emb, lin_W[0], lin_b[0][None])
    g, selfv, deg, dis = _t0b(hh, deg_parts, root_emb[0][None])
    for l in (0, 1):
        aggr = _s_edge(g, row3, col3, z128)
        g, selfv = _tmid(aggr, selfv, dis, deg,
                         (bn_gamma[l] * bn_scale)[None], bn_beta[l][None],
                         lin_W[l + 1], lin_b[l + 1][None],
                         root_emb[l + 1][None])
    aggr = _s_edge(g, row3, col3, z128)
    h3 = _t3(aggr, selfv, dis)
    pooled_p, cnt_p = _s_pool(h3, batch3, z128, zcol, onesc)
    return _t4(pooled_p, cnt_p, W1, b1[None], W2, b2[None])
